# Initial kernel scaffold; baseline (speedup 1.0000x reference)
import jax
import jax.numpy as jnp
from jax.experimental import pallas as pl


def kernel(x, pos_table, ln_gamma, ln_beta):
    raise NotImplementedError("write your pallas kernel here")


# speedup vs baseline: 4.9199x; 4.9199x over previous
import functools

import jax
import jax.numpy as jnp
from jax import lax
from jax.experimental import pallas as pl
from jax.experimental.pallas import tpu as pltpu
from jax.experimental.pallas import tpu_sc as plsc

_LANES = 16
_CH = 8


@functools.partial(jax.jit, static_argnums=(1,))
def _rowsum_sc(pos_table, s_rows):
    info = plsc.get_sparse_core_info()
    nc, ns = info.num_cores, info.num_subcores
    nw = nc * ns
    d = pos_table.shape[1]
    rows_w = s_rows // nw
    nch = rows_w // _CH
    mesh = plsc.VectorSubcoreMesh(core_axis_name="c", subcore_axis_name="s")

    @functools.partial(
        pl.kernel,
        mesh=mesh,
        out_type=jax.ShapeDtypeStruct((s_rows, _LANES), jnp.float32),
        scratch_types=[
            pltpu.VMEM((_CH, d), jnp.float32),
            pltpu.VMEM((_CH, d), jnp.float32),
            pltpu.VMEM((rows_w, _LANES), jnp.float32),
            pltpu.SemaphoreType.DMA,
            pltpu.SemaphoreType.DMA,
        ],
    )
    def rowsum(table_hbm, out_hbm, buf0, buf1, sums_v, sem0, sem1):
        wid = lax.axis_index("s") * nc + lax.axis_index("c")
        base = wid * rows_w
        bufs = (buf0, buf1)
        sems = (sem0, sem1)
        handles = [None, None]
        handles[0] = pltpu.async_copy(
            table_hbm.at[pl.ds(base, _CH)], buf0, sem0)
        for c in range(nch):
            k = c & 1
            handles[k].wait()
            if c + 1 < nch:
                handles[1 - k] = pltpu.async_copy(
                    table_hbm.at[pl.ds(base + (c + 1) * _CH, _CH)],
                    bufs[1 - k], sems[1 - k])
            buf = bufs[k]

            def row_body(r, _, buf=buf, c=c):
                def inner(j, accs):
                    a0, a1, a2, a3 = accs
                    col = j * 64
                    a0 = a0 + buf[r, pl.ds(col, _LANES)]
                    a1 = a1 + buf[r, pl.ds(col + 16, _LANES)]
                    a2 = a2 + buf[r, pl.ds(col + 32, _LANES)]
                    a3 = a3 + buf[r, pl.ds(col + 48, _LANES)]
                    return (a0, a1, a2, a3)

                z = jnp.zeros((_LANES,), jnp.float32)
                a0, a1, a2, a3 = lax.fori_loop(0, d // 64, inner, (z, z, z, z))
                sums_v[c * _CH + r, :] = (a0 + a1) + (a2 + a3)
                return 0

            lax.fori_loop(0, _CH, row_body, 0)
        pltpu.sync_copy(sums_v, out_hbm.at[pl.ds(base, rows_w)])

    return rowsum(pos_table)


def _ln_body(x_ref, p_ref, g_ref, b_ref, o_ref):
    d = p_ref.shape[1]
    pe_mean = jnp.sum(p_ref[...], axis=0, keepdims=True) * (1.0 / d)
    y = x_ref[...] + pe_mean
    mu = jnp.mean(y, axis=1, keepdims=True)
    dev = y - mu
    var = jnp.mean(dev * dev, axis=1, keepdims=True)
    o_ref[...] = dev * lax.rsqrt(var + 1e-5) * g_ref[...] + b_ref[...]


def kernel(x, pos_table, ln_gamma, ln_beta):
    b, s = x.shape
    partial = _rowsum_sc(pos_table, s)
    partial_t = partial.T
    g = ln_gamma.reshape(1, s)
    bt = ln_beta.reshape(1, s)
    return pl.pallas_call(
        _ln_body,
        out_shape=jax.ShapeDtypeStruct((b, s), jnp.float32),
    )(x, partial_t, g, bt)


# speedup vs baseline: 5.8261x; 1.1842x over previous
import functools

import jax
import jax.numpy as jnp
from jax import lax
from jax.experimental import pallas as pl
from jax.experimental.pallas import tpu as pltpu
from jax.experimental.pallas import tpu_sc as plsc

_LANES = 16
_CH = 8
_NBUF = 3


@functools.partial(jax.jit, static_argnums=(1,))
def _rowsum_sc(pos_table, s_rows):
    info = plsc.get_sparse_core_info()
    nc, ns = info.num_cores, info.num_subcores
    nw = nc * ns
    d = pos_table.shape[1]
    rows_w = s_rows // nw
    nch = rows_w // _CH
    mesh = plsc.VectorSubcoreMesh(core_axis_name="c", subcore_axis_name="s")

    @functools.partial(
        pl.kernel,
        mesh=mesh,
        out_type=jax.ShapeDtypeStruct((s_rows, _LANES), jnp.float32),
        scratch_types=[
            pltpu.VMEM((_NBUF, _CH, d), jnp.float32),
            pltpu.VMEM((rows_w, _LANES), jnp.float32),
        ] + [pltpu.SemaphoreType.DMA] * _NBUF,
    )
    def rowsum(table_hbm, out_hbm, bufs, sums_v, *sems):
        wid = lax.axis_index("s") * nc + lax.axis_index("c")
        base = wid * rows_w
        handles = [None] * _NBUF
        for p in range(min(_NBUF - 1, nch)):
            handles[p] = pltpu.async_copy(
                table_hbm.at[pl.ds(base + p * _CH, _CH)], bufs.at[p], sems[p])
        for c in range(nch):
            k = c % _NBUF
            handles[k].wait()
            nxt = c + _NBUF - 1
            if nxt < nch:
                handles[nxt % _NBUF] = pltpu.async_copy(
                    table_hbm.at[pl.ds(base + nxt * _CH, _CH)],
                    bufs.at[nxt % _NBUF], sems[nxt % _NBUF])

            @plsc.parallel_loop(0, _CH)
            def row_loop(r, k=k, c=c):
                buf = bufs.at[k]

                def inner(j, accs):
                    col = j * 128
                    return tuple(
                        accs[t] + buf[r, pl.ds(col + 16 * t, _LANES)]
                        for t in range(8))

                z = jnp.zeros((_LANES,), jnp.float32)
                accs = lax.fori_loop(0, d // 128, inner, (z,) * 8)
                acc = (((accs[0] + accs[1]) + (accs[2] + accs[3]))
                       + ((accs[4] + accs[5]) + (accs[6] + accs[7])))
                sums_v[c * _CH + r, :] = acc

        pltpu.sync_copy(sums_v, out_hbm.at[pl.ds(base, rows_w)])

    return rowsum(pos_table)


def _ln_body(x_ref, p_ref, g_ref, b_ref, o_ref):
    d = p_ref.shape[1]
    pe_mean = jnp.sum(p_ref[...], axis=0, keepdims=True) * (1.0 / d)
    y = x_ref[...] + pe_mean
    mu = jnp.mean(y, axis=1, keepdims=True)
    dev = y - mu
    var = jnp.mean(dev * dev, axis=1, keepdims=True)
    o_ref[...] = dev * lax.rsqrt(var + 1e-5) * g_ref[...] + b_ref[...]


def kernel(x, pos_table, ln_gamma, ln_beta):
    b, s = x.shape
    partial_t = _rowsum_sc(pos_table, s).T
    g = ln_gamma.reshape(1, s)
    bt = ln_beta.reshape(1, s)
    return pl.pallas_call(
        _ln_body,
        out_shape=jax.ShapeDtypeStruct((b, s), jnp.float32),
    )(x, partial_t, g, bt)


# speedup vs baseline: 6.8104x; 1.1689x over previous
import functools

import jax
import jax.numpy as jnp
from jax import lax
from jax.experimental import pallas as pl
from jax.experimental.pallas import tpu as pltpu
from jax.experimental.pallas import tpu_sc as plsc

_LANES = 16
_CH = 8
_NBUF = 3
_S_SC = 512
_TC_BLOCK = 256


@functools.partial(jax.jit, static_argnums=(1,))
def _rowsum_sc(pos_table, s_rows):
    info = plsc.get_sparse_core_info()
    nc, ns = info.num_cores, info.num_subcores
    nw = nc * ns
    d = pos_table.shape[1]
    rows_w = s_rows // nw
    nch = rows_w // _CH
    mesh = plsc.VectorSubcoreMesh(core_axis_name="c", subcore_axis_name="s")

    @functools.partial(
        pl.kernel,
        mesh=mesh,
        out_type=jax.ShapeDtypeStruct((s_rows, _LANES), jnp.float32),
        scratch_types=[
            pltpu.VMEM((_NBUF, _CH, d), jnp.float32),
            pltpu.VMEM((rows_w, _LANES), jnp.float32),
        ] + [pltpu.SemaphoreType.DMA] * _NBUF,
    )
    def rowsum(table_hbm, out_hbm, bufs, sums_v, *sems):
        wid = lax.axis_index("s") * nc + lax.axis_index("c")
        base = wid * rows_w
        handles = [None] * _NBUF
        for p in range(min(_NBUF - 1, nch)):
            handles[p] = pltpu.async_copy(
                table_hbm.at[pl.ds(base + p * _CH, _CH)], bufs.at[p], sems[p])
        for c in range(nch):
            k = c % _NBUF
            handles[k].wait()
            nxt = c + _NBUF - 1
            if nxt < nch:
                handles[nxt % _NBUF] = pltpu.async_copy(
                    table_hbm.at[pl.ds(base + nxt * _CH, _CH)],
                    bufs.at[nxt % _NBUF], sems[nxt % _NBUF])

            @plsc.parallel_loop(0, _CH)
            def row_loop(r, k=k, c=c):
                buf = bufs.at[k]

                def inner(j, accs):
                    col = j * 128
                    return tuple(
                        accs[t] + buf[r, pl.ds(col + 16 * t, _LANES)]
                        for t in range(8))

                z = jnp.zeros((_LANES,), jnp.float32)
                accs = lax.fori_loop(0, d // 128, inner, (z,) * 8)
                acc = (((accs[0] + accs[1]) + (accs[2] + accs[3]))
                       + ((accs[4] + accs[5]) + (accs[6] + accs[7])))
                sums_v[c * _CH + r, :] = acc

        pltpu.sync_copy(sums_v, out_hbm.at[pl.ds(base, rows_w)])

    return rowsum(pos_table)


def _rowsum_tc_body(t_ref, o_ref):
    d = t_ref.shape[1]
    ones = jnp.ones((1, d), jnp.float32)
    o_ref[...] = lax.dot_general(
        ones, t_ref[...], (((1,), (1,)), ((), ())),
        preferred_element_type=jnp.float32)


@functools.partial(jax.jit, static_argnums=(1, 2))
def _rowsum_tc(pos_table, lo, hi):
    d = pos_table.shape[1]
    nb = (hi - lo) // _TC_BLOCK
    return pl.pallas_call(
        _rowsum_tc_body,
        grid=(nb,),
        in_specs=[pl.BlockSpec((_TC_BLOCK, d),
                               lambda i: (i + lo // _TC_BLOCK, 0))],
        out_specs=pl.BlockSpec((1, _TC_BLOCK), lambda i: (0, i)),
        out_shape=jax.ShapeDtypeStruct((1, hi - lo), jnp.float32),
    )(pos_table)


def _ln_body(x_ref, pa_ref, pb_ref, g_ref, b_ref, o_ref):
    d = x_ref.shape[1]
    ones = jnp.ones((1, _LANES), jnp.float32)
    sa = lax.dot_general(
        ones, pa_ref[...], (((1,), (1,)), ((), ())),
        preferred_element_type=jnp.float32)
    pe_mean = jnp.concatenate([sa, pb_ref[...]], axis=1) * (1.0 / d)
    y = x_ref[...] + pe_mean
    mu = jnp.mean(y, axis=1, keepdims=True)
    dev = y - mu
    var = jnp.mean(dev * dev, axis=1, keepdims=True)
    o_ref[...] = dev * lax.rsqrt(var + 1e-5) * g_ref[...] + b_ref[...]


def kernel(x, pos_table, ln_gamma, ln_beta):
    b, s = x.shape
    partial_a = _rowsum_sc(pos_table, _S_SC)
    sums_b = _rowsum_tc(pos_table, _S_SC, s)
    g = ln_gamma.reshape(1, s)
    bt = ln_beta.reshape(1, s)
    return pl.pallas_call(
        _ln_body,
        out_shape=jax.ShapeDtypeStruct((b, s), jnp.float32),
    )(x, partial_a, sums_b, g, bt)


# speedup vs baseline: 6.8597x; 1.0072x over previous
import functools

import jax
import jax.numpy as jnp
from jax import lax
from jax.experimental import pallas as pl
from jax.experimental.pallas import tpu as pltpu
from jax.experimental.pallas import tpu_sc as plsc

_LANES = 16
_CH = 8
_NBUF = 3
_S_SC = 2048
_TC_BLOCK = 512


@functools.partial(jax.jit, static_argnums=(1,))
def _rowsum_sc(pos_table, s_rows):
    info = plsc.get_sparse_core_info()
    nc, ns = info.num_cores, info.num_subcores
    nw = nc * ns
    d = pos_table.shape[1]
    rows_w = s_rows // nw
    nch = rows_w // _CH
    mesh = plsc.VectorSubcoreMesh(core_axis_name="c", subcore_axis_name="s")

    @functools.partial(
        pl.kernel,
        mesh=mesh,
        out_type=jax.ShapeDtypeStruct((s_rows, _LANES), jnp.float32),
        scratch_types=[
            pltpu.VMEM((_NBUF, _CH, d), jnp.float32),
            pltpu.VMEM((rows_w, _LANES), jnp.float32),
        ] + [pltpu.SemaphoreType.DMA] * _NBUF,
    )
    def rowsum(table_hbm, out_hbm, bufs, sums_v, *sems):
        wid = lax.axis_index("s") * nc + lax.axis_index("c")
        base = wid * rows_w
        handles = [None] * _NBUF
        for p in range(min(_NBUF - 1, nch)):
            handles[p] = pltpu.async_copy(
                table_hbm.at[pl.ds(base + p * _CH, _CH)], bufs.at[p], sems[p])
        for c in range(nch):
            k = c % _NBUF
            handles[k].wait()
            nxt = c + _NBUF - 1
            if nxt < nch:
                handles[nxt % _NBUF] = pltpu.async_copy(
                    table_hbm.at[pl.ds(base + nxt * _CH, _CH)],
                    bufs.at[nxt % _NBUF], sems[nxt % _NBUF])

            @plsc.parallel_loop(0, _CH)
            def row_loop(r, k=k, c=c):
                buf = bufs.at[k]

                def inner(j, accs):
                    col = j * 128
                    return tuple(
                        accs[t] + buf[r, pl.ds(col + 16 * t, _LANES)]
                        for t in range(8))

                z = jnp.zeros((_LANES,), jnp.float32)
                accs = lax.fori_loop(0, d // 128, inner, (z,) * 8)
                acc = (((accs[0] + accs[1]) + (accs[2] + accs[3]))
                       + ((accs[4] + accs[5]) + (accs[6] + accs[7])))
                sums_v[c * _CH + r, :] = acc

        pltpu.sync_copy(sums_v, out_hbm.at[pl.ds(base, rows_w)])

    return rowsum(pos_table)


def _rowsum_tc_body(t_ref, o_ref):
    d = t_ref.shape[1]
    ones = jnp.ones((1, d), jnp.float32)
    o_ref[...] = lax.dot_general(
        ones, t_ref[...], (((1,), (1,)), ((), ())),
        preferred_element_type=jnp.float32)


@functools.partial(jax.jit, static_argnums=(1, 2))
def _rowsum_tc(pos_table, lo, hi):
    d = pos_table.shape[1]
    nb = (hi - lo) // _TC_BLOCK
    return pl.pallas_call(
        _rowsum_tc_body,
        grid=(nb,),
        in_specs=[pl.BlockSpec((_TC_BLOCK, d),
                               lambda i: (i + lo // _TC_BLOCK, 0))],
        out_specs=pl.BlockSpec((1, _TC_BLOCK), lambda i: (0, i)),
        out_shape=jax.ShapeDtypeStruct((1, hi - lo), jnp.float32),
    )(pos_table)


def _ln_body(x_ref, pa_ref, pb_ref, g_ref, b_ref, o_ref):
    d = x_ref.shape[1]
    ones = jnp.ones((1, _LANES), jnp.float32)
    sa = lax.dot_general(
        ones, pa_ref[...], (((1,), (1,)), ((), ())),
        preferred_element_type=jnp.float32)
    pe_mean = jnp.concatenate([sa, pb_ref[...]], axis=1) * (1.0 / d)
    y = x_ref[...] + pe_mean
    mu = jnp.mean(y, axis=1, keepdims=True)
    dev = y - mu
    var = jnp.mean(dev * dev, axis=1, keepdims=True)
    o_ref[...] = dev * lax.rsqrt(var + 1e-5) * g_ref[...] + b_ref[...]


def kernel(x, pos_table, ln_gamma, ln_beta):
    b, s = x.shape
    partial_a = _rowsum_sc(pos_table, _S_SC)
    sums_b = _rowsum_tc(pos_table, _S_SC, s)
    g = ln_gamma.reshape(1, s)
    bt = ln_beta.reshape(1, s)
    return pl.pallas_call(
        _ln_body,
        out_shape=jax.ShapeDtypeStruct((b, s), jnp.float32),
    )(x, partial_a, sums_b, g, bt)


# speedup vs baseline: 7.1500x; 1.0423x over previous
import functools

import jax
import jax.numpy as jnp
from jax import lax
from jax.experimental import pallas as pl
from jax.experimental.pallas import tpu as pltpu
from jax.experimental.pallas import tpu_sc as plsc

_LANES = 16
_CH = 8
_NBUF = 3
_S_SC = 1792
_TC_BLOCK = 512


@functools.partial(jax.jit, static_argnums=(1,))
def _rowsum_sc(pos_table, s_rows):
    info = plsc.get_sparse_core_info()
    nc, ns = info.num_cores, info.num_subcores
    nw = nc * ns
    d = pos_table.shape[1]
    rows_w = s_rows // nw
    nch = rows_w // _CH
    mesh = plsc.VectorSubcoreMesh(core_axis_name="c", subcore_axis_name="s")

    @functools.partial(
        pl.kernel,
        mesh=mesh,
        out_type=jax.ShapeDtypeStruct((s_rows, _LANES), jnp.float32),
        scratch_types=[
            pltpu.VMEM((_NBUF, _CH, d), jnp.float32),
            pltpu.VMEM((rows_w, _LANES), jnp.float32),
        ] + [pltpu.SemaphoreType.DMA] * _NBUF,
    )
    def rowsum(table_hbm, out_hbm, bufs, sums_v, *sems):
        wid = lax.axis_index("s") * nc + lax.axis_index("c")
        base = wid * rows_w
        handles = [None] * _NBUF
        for p in range(min(_NBUF - 1, nch)):
            handles[p] = pltpu.async_copy(
                table_hbm.at[pl.ds(base + p * _CH, _CH)], bufs.at[p], sems[p])
        for c in range(nch):
            k = c % _NBUF
            handles[k].wait()
            nxt = c + _NBUF - 1
            if nxt < nch:
                handles[nxt % _NBUF] = pltpu.async_copy(
                    table_hbm.at[pl.ds(base + nxt * _CH, _CH)],
                    bufs.at[nxt % _NBUF], sems[nxt % _NBUF])

            @plsc.parallel_loop(0, _CH)
            def row_loop(r, k=k, c=c):
                buf = bufs.at[k]

                def inner(j, accs):
                    col = j * 128
                    return tuple(
                        accs[t] + buf[r, pl.ds(col + 16 * t, _LANES)]
                        for t in range(8))

                z = jnp.zeros((_LANES,), jnp.float32)
                accs = lax.fori_loop(0, d // 128, inner, (z,) * 8)
                acc = (((accs[0] + accs[1]) + (accs[2] + accs[3]))
                       + ((accs[4] + accs[5]) + (accs[6] + accs[7])))
                sums_v[c * _CH + r, :] = acc

        pltpu.sync_copy(sums_v, out_hbm.at[pl.ds(base, rows_w)])

    return rowsum(pos_table)


def _rowsum_tc_body(t_ref, o_ref):
    d = t_ref.shape[1]
    ones = jnp.ones((1, d), jnp.float32)
    o_ref[...] = lax.dot_general(
        ones, t_ref[...], (((1,), (1,)), ((), ())),
        preferred_element_type=jnp.float32)


@functools.partial(jax.jit, static_argnums=(1, 2))
def _rowsum_tc(pos_table, lo, hi):
    d = pos_table.shape[1]
    nb = (hi - lo) // _TC_BLOCK
    return pl.pallas_call(
        _rowsum_tc_body,
        grid=(nb,),
        in_specs=[pl.BlockSpec((_TC_BLOCK, d),
                               lambda i: (i + lo // _TC_BLOCK, 0))],
        out_specs=pl.BlockSpec((1, _TC_BLOCK), lambda i: (0, i)),
        out_shape=jax.ShapeDtypeStruct((1, hi - lo), jnp.float32),
    )(pos_table)


def _ln_body(x_ref, pa_ref, pb_ref, g_ref, b_ref, o_ref):
    d = x_ref.shape[1]
    ones = jnp.ones((1, _LANES), jnp.float32)
    sa = lax.dot_general(
        ones, pa_ref[...], (((1,), (1,)), ((), ())),
        preferred_element_type=jnp.float32)
    pe_mean = jnp.concatenate([sa, pb_ref[...]], axis=1) * (1.0 / d)
    y = x_ref[...] + pe_mean
    mu = jnp.mean(y, axis=1, keepdims=True)
    dev = y - mu
    var = jnp.mean(dev * dev, axis=1, keepdims=True)
    o_ref[...] = dev * lax.rsqrt(var + 1e-5) * g_ref[...] + b_ref[...]


def kernel(x, pos_table, ln_gamma, ln_beta):
    b, s = x.shape
    partial_a = _rowsum_sc(pos_table, _S_SC)
    sums_b = _rowsum_tc(pos_table, _S_SC, s)
    g = ln_gamma.reshape(1, s)
    bt = ln_beta.reshape(1, s)
    return pl.pallas_call(
        _ln_body,
        out_shape=jax.ShapeDtypeStruct((b, s), jnp.float32),
    )(x, partial_a, sums_b, g, bt)
